# Initial kernel scaffold; baseline (speedup 1.0000x reference)
import jax
import jax.numpy as jnp
from jax.experimental import pallas as pl


def kernel(state, input_tails, exit_heads, unit_heads, unit_tails, unit_W, unit_b, step):
    raise NotImplementedError("write your pallas kernel here")


# speedup vs baseline: 2.6991x; 2.6991x over previous
import functools

import jax
import jax.numpy as jnp
import numpy as np
from jax.experimental import pallas as pl
from jax.experimental.pallas import tpu as pltpu

INPUT_SIZE = 16384
OUTPUT_SIZE = 1024
N_UNITS = 64
CHANNEL_SIZE = 64
ENV_DIMS = 32
REACH = 1.0
GARBAGE_DECAY = 0.9
REACH_THRESHOLD = float(np.clip(1.0 - REACH, -1.0, 1.0))
GARBAGE_SCALE = float(np.clip(1.0 - GARBAGE_DECAY, 0.0, 1.0))
CAPACITY = INPUT_SIZE + N_UNITS
N_QUERIES = N_UNITS + OUTPUT_SIZE

TILE = 2048
NTILES = INPUT_SIZE // TILE


def _normalize(x, eps=1e-12):
    n = jnp.sqrt(jnp.sum(x * x, axis=1, keepdims=True))
    return x / jnp.clip(n, eps, None)


def _fused_kernel(state_ref, tails_ref, eh_ref, uh_ref, w_ref, b_ref,
                  db_ref, exit_ref, gsum_ref, unit_ref,
                  dist_acc, gsum_acc):
    i = pl.program_id(0)

    q = jnp.concatenate([_normalize(uh_ref[...]), _normalize(eh_ref[...])],
                        axis=0)

    z = _normalize(_normalize(tails_ref[...]))

    sims = jnp.dot(q, z.T, preferred_element_type=jnp.float32)
    infl = jnp.maximum(sims - REACH_THRESHOLD, 0.0)

    st = state_ref[...]
    dist_part = jnp.dot(infl, st, preferred_element_type=jnp.float32)

    colsum = jnp.sum(infl, axis=0)
    t = st * (colsum - 1.0)[:, None]
    db_ref[...] = st - GARBAGE_SCALE * t
    g_part = -jnp.sum(t, axis=0, keepdims=True)

    @pl.when(i == 0)
    def _init():
        dist_acc[...] = dist_part
        gsum_acc[...] = g_part

    @pl.when(i > 0)
    def _accum():
        dist_acc[...] += dist_part
        gsum_acc[...] += g_part

    @pl.when(i == NTILES - 1)
    def _finish():
        dist = dist_acc[...]
        exit_ref[...] = dist[N_UNITS:]
        gsum_ref[...] = gsum_acc[...]
        unit_in = dist[:N_UNITS]
        w = w_ref[...]
        prod = jnp.sum(unit_in[:, :, None] * w, axis=1)
        unit_ref[...] = unit_in + jnp.maximum(prod + b_ref[...], 0.0)


@jax.jit
def _run(state, input_tails, exit_heads, unit_heads, unit_W, unit_b):
    out_shapes = (
        jax.ShapeDtypeStruct((INPUT_SIZE, CHANNEL_SIZE), jnp.float32),
        jax.ShapeDtypeStruct((OUTPUT_SIZE, CHANNEL_SIZE), jnp.float32),
        jax.ShapeDtypeStruct((1, CHANNEL_SIZE), jnp.float32),
        jax.ShapeDtypeStruct((N_UNITS, CHANNEL_SIZE), jnp.float32),
    )
    grid = (NTILES,)
    db_main, exit_out, gsum, unit_out = pl.pallas_call(
        _fused_kernel,
        grid=grid,
        in_specs=[
            pl.BlockSpec((TILE, CHANNEL_SIZE), lambda i: (i, 0)),
            pl.BlockSpec((TILE, ENV_DIMS), lambda i: (i, 0)),
            pl.BlockSpec((OUTPUT_SIZE, ENV_DIMS), lambda i: (0, 0)),
            pl.BlockSpec((N_UNITS, ENV_DIMS), lambda i: (0, 0)),
            pl.BlockSpec((N_UNITS, CHANNEL_SIZE, CHANNEL_SIZE),
                         lambda i: (0, 0, 0)),
            pl.BlockSpec((N_UNITS, CHANNEL_SIZE), lambda i: (0, 0)),
        ],
        out_specs=[
            pl.BlockSpec((TILE, CHANNEL_SIZE), lambda i: (i, 0)),
            pl.BlockSpec((OUTPUT_SIZE, CHANNEL_SIZE), lambda i: (0, 0)),
            pl.BlockSpec((1, CHANNEL_SIZE), lambda i: (0, 0)),
            pl.BlockSpec((N_UNITS, CHANNEL_SIZE), lambda i: (0, 0)),
        ],
        out_shape=out_shapes,
        scratch_shapes=[
            pltpu.VMEM((N_QUERIES, CHANNEL_SIZE), jnp.float32),
            pltpu.VMEM((1, CHANNEL_SIZE), jnp.float32),
        ],
    )(state, input_tails, exit_heads, unit_heads, unit_W, unit_b)
    db_data = jnp.concatenate([db_main, unit_out], axis=0)
    return exit_out, gsum.reshape(CHANNEL_SIZE), db_data


def kernel(state, input_tails, exit_heads, unit_heads, unit_tails, unit_W,
           unit_b, step=1):
    del unit_tails, step
    return _run(state, input_tails, exit_heads, unit_heads, unit_W, unit_b)


# speedup vs baseline: 3.2037x; 1.1869x over previous
import jax
import jax.numpy as jnp
import numpy as np
from jax.experimental import pallas as pl
from jax.experimental.pallas import tpu as pltpu

INPUT_SIZE = 16384
OUTPUT_SIZE = 1024
N_UNITS = 64
CHANNEL_SIZE = 64
ENV_DIMS = 32
REACH = 1.0
GARBAGE_DECAY = 0.9
REACH_THRESHOLD = float(np.clip(1.0 - REACH, -1.0, 1.0))
GARBAGE_SCALE = float(np.clip(1.0 - GARBAGE_DECAY, 0.0, 1.0))
CAPACITY = INPUT_SIZE + N_UNITS
N_QUERIES = N_UNITS + OUTPUT_SIZE

TILE = CAPACITY // 8
NTILES = 8


def _normalize(x):
    s = jnp.sum(x * x, axis=1, keepdims=True)
    return x * jax.lax.rsqrt(jnp.clip(s, 1e-24, None))


def _fused_kernel(state_ref, tails_ref, eh_ref, uh_ref, w_ref, b_ref,
                  db_ref, exit_ref, gsum_ref,
                  q_ref, dist_acc, gsum_acc):
    i = pl.program_id(0)

    @pl.when(i == 0)
    def _init_queries():
        q_ref[...] = jnp.concatenate(
            [_normalize(uh_ref[...]), _normalize(eh_ref[...])], axis=0)

    rows = jax.lax.broadcasted_iota(jnp.int32, (TILE, 1), 0) + i * TILE
    valid = rows < INPUT_SIZE
    z = _normalize(jnp.where(valid, tails_ref[...], 0.0))
    st = jnp.where(valid, state_ref[...], 0.0)

    sims = jax.lax.dot_general(
        q_ref[...], z, (((1,), (1,)), ((), ())),
        preferred_element_type=jnp.float32)
    infl = jnp.maximum(sims - REACH_THRESHOLD, 0.0)

    dist_part = jnp.dot(infl, st, preferred_element_type=jnp.float32)

    colsum = jnp.sum(infl, axis=0)
    t = st * (colsum - 1.0)[:, None]
    db_ref[...] = st - GARBAGE_SCALE * t
    g_part = -jnp.sum(t, axis=0, keepdims=True)

    @pl.when(i == 0)
    def _init():
        dist_acc[...] = dist_part
        gsum_acc[...] = g_part

    @pl.when(i > 0)
    def _accum():
        dist_acc[...] += dist_part
        gsum_acc[...] += g_part

    @pl.when(i == NTILES - 1)
    def _finish():
        dist = dist_acc[...]
        exit_ref[...] = dist[N_UNITS:]
        gsum_ref[...] = gsum_acc[...]
        unit_in = dist[:N_UNITS]
        w = w_ref[...]
        prod = jnp.sum(unit_in[:, :, None] * w, axis=1)
        unit_out = unit_in + jnp.maximum(prod + b_ref[...], 0.0)
        db_ref[TILE - N_UNITS:, :] = unit_out


@jax.jit
def _run(state, input_tails, exit_heads, unit_heads, unit_W, unit_b):
    out_shapes = (
        jax.ShapeDtypeStruct((CAPACITY, CHANNEL_SIZE), jnp.float32),
        jax.ShapeDtypeStruct((OUTPUT_SIZE, CHANNEL_SIZE), jnp.float32),
        jax.ShapeDtypeStruct((1, CHANNEL_SIZE), jnp.float32),
    )
    db_data, exit_out, gsum = pl.pallas_call(
        _fused_kernel,
        grid=(NTILES,),
        in_specs=[
            pl.BlockSpec((TILE, CHANNEL_SIZE), lambda i: (i, 0)),
            pl.BlockSpec((TILE, ENV_DIMS), lambda i: (i, 0)),
            pl.BlockSpec((OUTPUT_SIZE, ENV_DIMS), lambda i: (0, 0)),
            pl.BlockSpec((N_UNITS, ENV_DIMS), lambda i: (0, 0)),
            pl.BlockSpec((N_UNITS, CHANNEL_SIZE, CHANNEL_SIZE),
                         lambda i: (0, 0, 0)),
            pl.BlockSpec((N_UNITS, CHANNEL_SIZE), lambda i: (0, 0)),
        ],
        out_specs=[
            pl.BlockSpec((TILE, CHANNEL_SIZE), lambda i: (i, 0)),
            pl.BlockSpec((OUTPUT_SIZE, CHANNEL_SIZE), lambda i: (0, 0)),
            pl.BlockSpec((1, CHANNEL_SIZE), lambda i: (0, 0)),
        ],
        out_shape=out_shapes,
        scratch_shapes=[
            pltpu.VMEM((N_QUERIES, ENV_DIMS), jnp.float32),
            pltpu.VMEM((N_QUERIES, CHANNEL_SIZE), jnp.float32),
            pltpu.VMEM((1, CHANNEL_SIZE), jnp.float32),
        ],
    )(state, input_tails, exit_heads, unit_heads, unit_W, unit_b)
    return exit_out, gsum.reshape(CHANNEL_SIZE), db_data


def kernel(state, input_tails, exit_heads, unit_heads, unit_tails, unit_W,
           unit_b, step=1):
    del unit_tails, step
    return _run(state, input_tails, exit_heads, unit_heads, unit_W, unit_b)
